# baseline (speedup 1.0000x reference)
import functools

import jax
import jax.numpy as jnp
from jax import lax
from jax.experimental import pallas as pl
from jax.experimental.pallas import tpu as pltpu
from jax.experimental.pallas import tpu_sc as plsc

N = 10000
E = 160000
D_IN = 128
D_EDGE = 16
D = 32
EH = 32
STEPS = 6

NC = 2
NS = 16
NW = NC * NS
CHUNK = 128
E_PAD = 163840
EPW = E_PAD // NW
CPW = EPW // CHUNK
N_PAD = 10240
RPW = N_PAD // NS

_sc_mesh = plsc.VectorSubcoreMesh(core_axis_name="c", subcore_axis_name="s")


@functools.partial(
    pl.kernel,
    mesh=_sc_mesh,
    compiler_params=pltpu.CompilerParams(use_tc_tiling_on_sc=False),
    out_type=jax.ShapeDtypeStruct((E_PAD, D), jnp.float32),
    scratch_types=[
        pltpu.VMEM((CPW, CHUNK), jnp.int32),
        pltpu.VMEM((CHUNK, D), jnp.float32),
        pltpu.SemaphoreType.DMA,
    ],
)
def _sc_gather(h_hbm, src3_hbm, t_hbm, idx_v, rows_v, sem):
    wid = lax.axis_index("s") * NC + lax.axis_index("c")
    base = wid * EPW
    pltpu.sync_copy(src3_hbm.at[wid], idx_v)
    for j in range(CPW):
        pltpu.async_copy(h_hbm.at[idx_v.at[j]], rows_v, sem).wait()
        pltpu.sync_copy(rows_v, t_hbm.at[pl.ds(base + j * CHUNK, CHUNK)])


@functools.partial(
    pl.kernel,
    mesh=_sc_mesh,
    compiler_params=pltpu.CompilerParams(use_tc_tiling_on_sc=False),
    out_type=jax.ShapeDtypeStruct((NC, N_PAD, D), jnp.float32),
    scratch_types=[
        pltpu.VMEM((CPW, CHUNK), jnp.int32),
        pltpu.VMEM((CHUNK, D), jnp.float32),
        pltpu.VMEM_SHARED((N_PAD, D), jnp.float32),
    ],
)
def _sc_scatter(m_hbm, dst3_hbm, zeros_hbm, out_hbm, idx_v, mbuf, agg_sh):
    cid = lax.axis_index("c")
    sid = lax.axis_index("s")
    wid = sid * NC + cid
    base = wid * EPW
    pltpu.sync_copy(dst3_hbm.at[wid], idx_v)

    @pl.when(sid == 0)
    def _():
        pltpu.sync_copy(zeros_hbm, agg_sh)

    plsc.subcore_barrier()
    for j in range(CPW):
        pltpu.sync_copy(m_hbm.at[pl.ds(base + j * CHUNK, CHUNK)], mbuf)
        pltpu.sync_copy(mbuf, agg_sh.at[idx_v.at[j]], add=True)
    plsc.subcore_barrier()
    pltpu.sync_copy(
        agg_sh.at[pl.ds(sid * RPW, RPW)],
        out_hbm.at[cid, pl.ds(sid * RPW, RPW)],
    )


def _proj_body(nf_ref, w_ref, b_ref, o_ref):
    acc = jnp.dot(nf_ref[...].astype(jnp.bfloat16),
                  w_ref[...].astype(jnp.bfloat16),
                  preferred_element_type=jnp.float32)
    o_ref[...] = jnp.maximum(acc + b_ref[...], 0.0)


def _proj(nf, w, b):
    tn = 1000
    return pl.pallas_call(
        _proj_body,
        grid=(N // tn,),
        in_specs=[
            pl.BlockSpec((tn, D_IN), lambda i: (i, 0)),
            pl.BlockSpec((D_IN, D), lambda i: (0, 0)),
            pl.BlockSpec((1, D), lambda i: (0, 0)),
        ],
        out_specs=pl.BlockSpec((tn, D), lambda i: (i, 0)),
        out_shape=jax.ShapeDtypeStruct((N, D), jnp.float32),
    )(nf, w, b)


def _we_body(ef_ref, w1_ref, b1_ref, w2_ref, b2_ref, o_ref):
    ef = ef_ref[...].astype(jnp.bfloat16)
    r = jnp.maximum(
        jnp.dot(ef, w1_ref[...].astype(jnp.bfloat16),
                preferred_element_type=jnp.float32) + b1_ref[...], 0.0)
    we = jnp.dot(
        r.astype(jnp.bfloat16), w2_ref[...].astype(jnp.bfloat16),
        preferred_element_type=jnp.float32) + b2_ref[...]
    o_ref[...] = we.astype(jnp.bfloat16)


def _we(ef_pad, w1, b1, w2, b2):
    te = 1024
    return pl.pallas_call(
        _we_body,
        grid=(E_PAD // te,),
        in_specs=[
            pl.BlockSpec((te, D_EDGE), lambda i: (i, 0)),
            pl.BlockSpec((D_EDGE, EH), lambda i: (0, 0)),
            pl.BlockSpec((1, EH), lambda i: (0, 0)),
            pl.BlockSpec((EH, D * D), lambda i: (0, 0)),
            pl.BlockSpec((1, D * D), lambda i: (0, 0)),
        ],
        out_specs=pl.BlockSpec((te, D * D), lambda i: (i, 0)),
        out_shape=jax.ShapeDtypeStruct((E_PAD, D * D), jnp.bfloat16),
    )(ef_pad, w1, b1, w2, b2)


_TE_MSG = 512


def _msg_body(t_ref, we_ref, o_ref):
    t = t_ref[...].astype(jnp.bfloat16).astype(jnp.float32)
    w = we_ref[...].astype(jnp.float32)
    m = t[:, 0:1] * w[:, 0:D]
    for i in range(1, D):
        m = m + t[:, i:i + 1] * w[:, i * D:(i + 1) * D]
    o_ref[...] = m


def _msg(t, we2):
    return pl.pallas_call(
        _msg_body,
        grid=(E_PAD // _TE_MSG,),
        in_specs=[
            pl.BlockSpec((_TE_MSG, D), lambda i: (i, 0)),
            pl.BlockSpec((_TE_MSG, D * D), lambda i: (i, 0)),
        ],
        out_specs=pl.BlockSpec((_TE_MSG, D), lambda i: (i, 0)),
        out_shape=jax.ShapeDtypeStruct((E_PAD, D), jnp.float32),
    )(t, we2)


def _gru_body(a0_ref, a1_ref, h_ref, cb_ref, wih_ref, bih_ref, whh_ref,
              bhh_ref, o_ref):
    x = jnp.maximum(a0_ref[...] + a1_ref[...] + cb_ref[...], 0.0)
    h = h_ref[...]
    gi = jnp.dot(x.astype(jnp.bfloat16), wih_ref[...].astype(jnp.bfloat16),
                 preferred_element_type=jnp.float32)
    gi = gi + bih_ref[...]
    gh = jnp.dot(h.astype(jnp.bfloat16), whh_ref[...].astype(jnp.bfloat16),
                 preferred_element_type=jnp.float32)
    gh = gh + bhh_ref[...]
    rg = jax.nn.sigmoid(gi[:, :D] + gh[:, :D])
    z = jax.nn.sigmoid(gi[:, D:2 * D] + gh[:, D:2 * D])
    n = jnp.tanh(gi[:, 2 * D:] + rg * gh[:, 2 * D:])
    o_ref[...] = (1.0 - z) * n + z * h


def _gru(a0, a1, h, cb, wih, bih, whh, bhh):
    tn = 1000
    return pl.pallas_call(
        _gru_body,
        grid=(N // tn,),
        in_specs=[
            pl.BlockSpec((tn, D), lambda i: (i, 0)),
            pl.BlockSpec((tn, D), lambda i: (i, 0)),
            pl.BlockSpec((tn, D), lambda i: (i, 0)),
            pl.BlockSpec((1, D), lambda i: (0, 0)),
            pl.BlockSpec((D, 3 * D), lambda i: (0, 0)),
            pl.BlockSpec((1, 3 * D), lambda i: (0, 0)),
            pl.BlockSpec((D, 3 * D), lambda i: (0, 0)),
            pl.BlockSpec((1, 3 * D), lambda i: (0, 0)),
        ],
        out_specs=pl.BlockSpec((tn, D), lambda i: (i, 0)),
        out_shape=jax.ShapeDtypeStruct((N, D), jnp.float32),
    )(a0, a1, h, cb, wih, bih, whh, bhh)


@jax.jit
def _mpnn(node_feats, edge_feats, edge_index, proj_w, proj_b, e1_w, e1_b,
          e2_w, e2_b, conv_b, gru_w_ih, gru_b_ih, gru_w_hh, gru_b_hh):
    src = edge_index[0].astype(jnp.int32)
    dst = edge_index[1].astype(jnp.int32)
    src3 = jnp.concatenate(
        [src, jnp.zeros((E_PAD - E,), jnp.int32)]).reshape(NW, CPW, CHUNK)
    dst3 = jnp.concatenate(
        [dst, jnp.full((E_PAD - E,), N, jnp.int32)]).reshape(NW, CPW, CHUNK)

    zeros = jnp.zeros((N_PAD, D), jnp.float32)
    ef_pad = jnp.pad(edge_feats, ((0, E_PAD - E), (0, 0)))

    h = _proj(node_feats, proj_w, proj_b.reshape(1, D))
    we2 = _we(ef_pad, e1_w, e1_b.reshape(1, EH), e2_w, e2_b.reshape(1, D * D))

    hidden = h
    for _ in range(STEPS):
        t = _sc_gather(h, src3)
        m = _msg(t, we2)
        agg2 = _sc_scatter(m, dst3, zeros)
        hidden = _gru(agg2[0, :N], agg2[1, :N], hidden,
                      conv_b.reshape(1, D),
                      gru_w_ih, gru_b_ih.reshape(1, 3 * D),
                      gru_w_hh, gru_b_hh.reshape(1, 3 * D))
        h = hidden
    return h


def kernel(node_feats, edge_feats, edge_index, proj_w, proj_b, e1_w, e1_b,
           e2_w, e2_b, conv_b, gru_w_ih, gru_b_ih, gru_w_hh, gru_b_hh):
    return _mpnn(node_feats, edge_feats, edge_index, proj_w, proj_b, e1_w,
                 e1_b, e2_w, e2_b, conv_b, gru_w_ih, gru_b_ih, gru_w_hh,
                 gru_b_hh)
